# baseline (speedup 1.0000x reference)
import functools

import jax
import jax.numpy as jnp
from jax import lax
from jax.experimental import pallas as pl
from jax.experimental.pallas import tpu as pltpu
from jax.experimental.pallas import tpu_sc as plsc

_NC, _NS = 2, 16
_NW = _NC * _NS
_BSZ, _SEQ = 4, 4096
_B = _BSZ * _SEQ
_BW = _B // _NW
_TPB = _SEQ // _BW
_D = 128

_mesh = plsc.VectorSubcoreMesh(core_axis_name="c", subcore_axis_name="s")


@functools.partial(
    pl.kernel,
    out_type=(
        jax.ShapeDtypeStruct((_BSZ, _SEQ, _D), jnp.float32),
        jax.ShapeDtypeStruct((_BSZ, _SEQ, _D), jnp.float32),
    ),
    mesh=_mesh,
    scratch_types=[
        pltpu.VMEM((_BW,), jnp.int32),
        [pltpu.VMEM((_BW // 2, _D), jnp.float32) for _ in range(3)],
        [pltpu.SemaphoreType.DMA for _ in range(3)],
        [pltpu.SemaphoreType.DMA for _ in range(3)],
    ],
)
def _rope_gather(cos_hbm, sin_hbm, idx_hbm, cos_out, sin_out,
                 idx_v, bufs, gsems, wsems):
    wid = lax.axis_index("s") * _NC + lax.axis_index("c")
    b = wid // _TPB
    s0 = (wid % _TPB) * _BW
    half = _BW // 2
    pltpu.sync_copy(idx_hbm.at[b, pl.ds(s0, _BW)], idx_v)

    def job(j):
        tbl, out = (cos_hbm, cos_out) if j < 2 else (sin_hbm, sin_out)
        return tbl, out, (j % 2) * half

    gathers, writes = [None] * 4, [None] * 4

    def start_gather(j):
        tbl, _, off = job(j)
        gathers[j] = pltpu.async_copy(
            tbl.at[idx_v.at[pl.ds(off, half)]], bufs[j % 3], gsems[j % 3])

    start_gather(0)
    start_gather(1)
    for j in range(4):
        if j + 2 < 4:
            if j + 2 >= 3:
                writes[j - 1].wait()
            start_gather(j + 2)
        _, out, off = job(j)
        gathers[j].wait()
        writes[j] = pltpu.async_copy(
            bufs[j % 3], out.at[b, pl.ds(s0 + off, half), :], wsems[j % 3])
    writes[1].wait()
    writes[2].wait()
    writes[3].wait()


def kernel(x, position_ids, cos_cached, sin_cached):
    idx = position_ids.astype(jnp.int32)
    cos, sin = _rope_gather(cos_cached, sin_cached, idx)
    return (cos.astype(x.dtype), sin.astype(x.dtype))


# speedup vs baseline: 1.0055x; 1.0055x over previous
import functools

import jax
import jax.numpy as jnp
from jax import lax
from jax.experimental import pallas as pl
from jax.experimental.pallas import tpu as pltpu
from jax.experimental.pallas import tpu_sc as plsc

_NC, _NS = 2, 16
_NW = _NC * _NS
_BSZ, _SEQ = 4, 4096
_B = _BSZ * _SEQ
_BW = _B // _NW
_TPB = _SEQ // _BW
_D = 128

_mesh = plsc.VectorSubcoreMesh(core_axis_name="c", subcore_axis_name="s")


@functools.partial(
    pl.kernel,
    out_type=(
        jax.ShapeDtypeStruct((_BSZ, _SEQ, _D), jnp.float32),
        jax.ShapeDtypeStruct((_BSZ, _SEQ, _D), jnp.float32),
    ),
    mesh=_mesh,
    scratch_types=[
        pltpu.VMEM((_BW,), jnp.int32),
        pltpu.VMEM((_BW, _D), jnp.float32),
        pltpu.SemaphoreType.DMA,
    ],
)
def _rope_gather(cos_hbm, sin_hbm, idx_hbm, cos_out, sin_out,
                 idx_v, rows_v, sem):
    wid = lax.axis_index("s") * _NC + lax.axis_index("c")
    b = wid // _TPB
    s0 = (wid % _TPB) * _BW
    rows = pl.ds(s0, _BW)
    pltpu.sync_copy(idx_hbm.at[b, rows], idx_v)
    pltpu.async_copy(cos_hbm.at[idx_v], rows_v, sem).wait()
    pltpu.sync_copy(rows_v, cos_out.at[b, rows, :])
    pltpu.async_copy(sin_hbm.at[idx_v], rows_v, sem).wait()
    pltpu.sync_copy(rows_v, sin_out.at[b, rows, :])


def kernel(x, position_ids, cos_cached, sin_cached):
    idx = position_ids.astype(jnp.int32)
    cos, sin = _rope_gather(cos_cached, sin_cached, idx)
    return (cos.astype(x.dtype), sin.astype(x.dtype))
